# baseline (speedup 1.0000x reference)
import functools

import jax
import jax.numpy as jnp
from jax import lax
from jax.experimental import pallas as pl
from jax.experimental.pallas import tpu as pltpu
from jax.experimental.pallas import tpu_sc as plsc

_N = 1_000_000
_L = 16
_NTILES = 16
_CHUNK = 62_528
_TAIL = _N - 15 * _CHUNK
_VPW = _CHUNK // _L
_VPT = _TAIL // _L
_WPW = _CHUNK // 64
_K = 120_000
_NB = 256
_HWORDS = _NB * _L
_CCAP = 4080

_mesh = plsc.VectorSubcoreMesh(
    core_axis_name="c", subcore_axis_name="s", num_cores=1
)


@functools.partial(
    pl.kernel,
    mesh=_mesh,
    out_type=jax.ShapeDtypeStruct((_N // 4,), jnp.int32),
    scratch_types=[
        pltpu.VMEM((_CHUNK,), jnp.float32),
        pltpu.VMEM((_HWORDS,), jnp.int32),
        pltpu.VMEM((_HWORDS,), jnp.int32),
        pltpu.VMEM((_CCAP + _L,), jnp.float32),
        pltpu.VMEM((_CHUNK // 4,), jnp.int32),
        pltpu.VMEM_SHARED((_NTILES, _HWORDS), jnp.int32),
        pltpu.SMEM((34,), jnp.int32),
    ],
    compiler_params=pltpu.CompilerParams(needs_layout_passes=False),
)
def _topk_mask(
    x_hbm, out_hbm, data_v, hist_v, tmp_v, cand_v, pack_v, hist_sh, cnt_sm
):
    tid = lax.axis_index("s")
    base = tid * _CHUNK
    nvec = jnp.where(tid == _NTILES - 1, _VPT, _VPW)

    @pl.when(tid == _NTILES - 1)
    def _load_tail():
        pltpu.sync_copy(x_hbm.at[pl.ds(base, _TAIL)], data_v.at[pl.ds(0, _TAIL)])

    @pl.when(tid != _NTILES - 1)
    def _load_full():
        pltpu.sync_copy(x_hbm.at[pl.ds(base, _CHUNK)], data_v)

    lanes = lax.iota(jnp.int32, _L)
    zeros_i = jnp.zeros((_L,), jnp.int32)
    zeros_f = jnp.zeros((_L,), jnp.float32)
    ones_i = jnp.ones((_L,), jnp.int32)

    @pl.when(tid == 0)
    def _zero_sm():
        def z(i, c):
            cnt_sm[i] = 0
            return c

        lax.fori_loop(0, 34, z, 0)

    @plsc.parallel_loop(0, _HWORDS // _L, unroll=8)
    def _zh(i):
        hist_v[pl.ds(i * _L, _L)] = zeros_i

    @plsc.parallel_loop(0, (_CCAP + _L) // _L, unroll=8)
    def _zc(i):
        cand_v[pl.ds(i * _L, _L)] = zeros_f

    @plsc.parallel_loop(0, nvec, unroll=8)
    def _hist(i):
        v = data_v[pl.ds(i * _L, _L)]
        idx = (v * 256.0).astype(jnp.int32) * _L + lanes
        plsc.addupdate_scatter(hist_v, [idx], ones_i)

    pltpu.sync_copy(hist_v, hist_sh.at[tid])
    plsc.subcore_barrier()

    def merge(s, c):
        peer = lax.rem(tid + s, _NTILES)
        pltpu.sync_copy(hist_sh.at[peer], tmp_v)

        @plsc.parallel_loop(0, _HWORDS // _L, unroll=8)
        def _acc(i):
            hist_v[pl.ds(i * _L, _L)] = (
                hist_v[pl.ds(i * _L, _L)] + tmp_v[pl.ds(i * _L, _L)]
            )

        return c

    lax.fori_loop(1, _NTILES, merge, 0)

    def scan_body(j, carry):
        s, b_win, above = carry
        b = _NB - 1 - j
        s_new = s + jnp.sum(hist_v[pl.ds(b * _L, _L)])
        hit = (s_new >= _K) & (s < _K)
        return (
            s_new,
            jnp.where(hit, b, b_win),
            jnp.where(hit, s, above),
        )

    _, b_win, above = plsc.parallel_loop(
        0, _NB, unroll=4,
        carry=(jnp.int32(0), jnp.int32(0), jnp.int32(0)),
    )(scan_body)
    kprime = _K - above
    bwin_v = jnp.full((_L,), b_win, dtype=jnp.int32)

    def compact_body(i, off_c):
        v = data_v[pl.ds(i * _L, _L)]
        m = (v * 256.0).astype(jnp.int32) == bwin_v
        m = m & jnp.full((_L,), off_c < _CCAP - _L, dtype=jnp.bool_)
        plsc.store_compressed(cand_v.at[pl.ds(off_c, _L)], v, mask=m)
        return off_c + plsc.all_reduce_population_count(m)[0]

    off = plsc.parallel_loop(0, nvec, unroll=8, carry=jnp.int32(0))(
        compact_body
    )
    nvc = (off + _L - 1) // _L

    def round_body(r, lohi):
        lo, hi = lohi
        mid = (lo + hi) // 2
        tvec = jnp.full(
            (_L,), lax.bitcast_convert_type(mid, jnp.float32), dtype=jnp.float32
        )

        def cnt(i, a):
            v = cand_v[pl.ds(i * _L, _L)]
            return a + plsc.all_reduce_population_count(v >= tvec)

        acc = lax.fori_loop(0, nvc, cnt, zeros_i)
        plsc.fetch_and_add(cnt_sm.at[r], acc[0], subcore_id=0)
        plsc.subcore_barrier()
        tot = plsc.fetch_and_add(cnt_sm.at[r], jnp.int32(0), subcore_id=0)
        ok = tot >= kprime
        return jnp.where(ok, mid, lo), jnp.where(ok, hi, mid)

    lo0 = jnp.where(
        b_win > 0,
        lax.bitcast_convert_type(b_win.astype(jnp.float32) * (1.0 / 256.0),
                                 jnp.int32),
        jnp.int32(0),
    )
    hi0 = lax.bitcast_convert_type(
        (b_win + 1).astype(jnp.float32) * (1.0 / 256.0), jnp.int32
    )
    nrounds = (
        lax.shift_right_logical(
            lax.bitcast_convert_type((hi0 - lo0).astype(jnp.float32), jnp.int32),
            23,
        )
        - 126
    )
    lo, _ = lax.fori_loop(0, nrounds, round_body, (lo0, hi0))
    tau = jnp.full(
        (_L,),
        lax.bitcast_convert_type(lo, jnp.float32) * 0.25,
        dtype=jnp.float32,
    )

    @plsc.parallel_loop(0, _WPW, unroll=4)
    def _pack(i):
        e0 = i * 64
        ma = jnp.where(data_v[pl.ds(e0, _L)] * 0.25 >= tau, 1, 0)
        mb = jnp.where(data_v[pl.ds(e0 + 16, _L)] * 0.25 >= tau, 1, 0)
        mc = jnp.where(data_v[pl.ds(e0 + 32, _L)] * 0.25 >= tau, 1, 0)
        md = jnp.where(data_v[pl.ds(e0 + 48, _L)] * 0.25 >= tau, 1, 0)
        w = ma | (mb << 8) | (mc << 16) | (md << 24)
        pack_v[pl.ds(i * _L, _L)] = w.astype(jnp.int32)

    wbase = tid * (_CHUNK // 4)

    @pl.when(tid == _NTILES - 1)
    def _store_tail():
        pltpu.sync_copy(
            pack_v.at[pl.ds(0, _TAIL // 4)],
            out_hbm.at[pl.ds(wbase, _TAIL // 4)],
        )

    @pl.when(tid != _NTILES - 1)
    def _store_full():
        pltpu.sync_copy(pack_v, out_hbm.at[pl.ds(wbase, _CHUNK // 4)])


def kernel(strength):
    x = strength.reshape(-1)
    m32 = _topk_mask(x)
    mb = lax.bitcast_convert_type(m32.reshape(_N // 64, 16), jnp.uint8)
    mb = mb.transpose(0, 2, 1).reshape(_N)
    return mb.astype(jnp.bool_).reshape(_N, 1)


# speedup vs baseline: 1.2917x; 1.2917x over previous
import functools

import jax
import jax.numpy as jnp
from jax import lax
from jax.experimental import pallas as pl
from jax.experimental.pallas import tpu as pltpu
from jax.experimental.pallas import tpu_sc as plsc

_N = 1_000_000
_L = 16
_NTILES = 16
_CHUNK = 62_528
_TAIL = _N - 15 * _CHUNK
_VPW = _CHUNK // _L
_VPT = _TAIL // _L
_K = 120_000
_NB = 256
_HWORDS = _NB * _L
_CCAP = 4080

_mesh = plsc.VectorSubcoreMesh(
    core_axis_name="c", subcore_axis_name="s", num_cores=1
)


@functools.partial(
    pl.kernel,
    mesh=_mesh,
    out_type=jax.ShapeDtypeStruct((_N,), jnp.float32),
    scratch_types=[
        pltpu.VMEM((_CHUNK,), jnp.float32),
        pltpu.VMEM((_HWORDS,), jnp.int32),
        pltpu.VMEM((_HWORDS,), jnp.int32),
        pltpu.VMEM((_HWORDS,), jnp.int32),
        pltpu.VMEM((_CCAP + _L,), jnp.float32),
        pltpu.VMEM_SHARED((_NTILES, _HWORDS), jnp.int32),
        pltpu.SMEM((34,), jnp.int32),
        pltpu.SemaphoreType.DMA,
    ],
    compiler_params=pltpu.CompilerParams(needs_layout_passes=False),
)
def _topk_mask(
    x_hbm, out_hbm, data_v, hist_v, tmp_v, tmpb_v, cand_v, hist_sh, cnt_sm,
    dma_sem,
):
    tid = lax.axis_index("s")
    base = tid * _CHUNK
    nvec = jnp.where(tid == _NTILES - 1, _VPT, _VPW)

    @pl.when(tid == _NTILES - 1)
    def _load_tail():
        pltpu.sync_copy(x_hbm.at[pl.ds(base, _TAIL)], data_v.at[pl.ds(0, _TAIL)])

    @pl.when(tid != _NTILES - 1)
    def _load_full():
        pltpu.sync_copy(x_hbm.at[pl.ds(base, _CHUNK)], data_v)

    lanes = lax.iota(jnp.int32, _L)
    zeros_i = jnp.zeros((_L,), jnp.int32)
    zeros_f = jnp.zeros((_L,), jnp.float32)
    ones_i = jnp.ones((_L,), jnp.int32)

    @pl.when(tid == 0)
    def _zero_sm():
        def z(i, c):
            cnt_sm[i] = 0
            return c

        lax.fori_loop(0, 34, z, 0)

    @plsc.parallel_loop(0, _HWORDS // _L, unroll=8)
    def _zh(i):
        hist_v[pl.ds(i * _L, _L)] = zeros_i

    @plsc.parallel_loop(0, (_CCAP + _L) // _L, unroll=8)
    def _zc(i):
        cand_v[pl.ds(i * _L, _L)] = zeros_f

    @plsc.parallel_loop(0, nvec, unroll=8)
    def _hist(i):
        v = data_v[pl.ds(i * _L, _L)]
        idx = (v * 256.0).astype(jnp.int32) * _L + lanes
        plsc.addupdate_scatter(hist_v, [idx], ones_i)

    pltpu.sync_copy(hist_v, hist_sh.at[tid])
    plsc.subcore_barrier()

    bufs = (tmp_v, tmpb_v)
    cp = pltpu.async_copy(
        hist_sh.at[lax.rem(tid + 1, _NTILES)], bufs[0], dma_sem
    )
    for s in range(1, _NTILES):
        cp.wait()
        cur = bufs[(s - 1) % 2]
        if s < _NTILES - 1:
            cp = pltpu.async_copy(
                hist_sh.at[lax.rem(tid + s + 1, _NTILES)],
                bufs[s % 2],
                dma_sem,
            )

        @plsc.parallel_loop(0, _HWORDS // _L, unroll=8)
        def _acc(i, cur=cur):
            hist_v[pl.ds(i * _L, _L)] = (
                hist_v[pl.ds(i * _L, _L)] + cur[pl.ds(i * _L, _L)]
            )

    def scan_body(j, carry):
        s, b_win, above = carry
        b = _NB - 1 - j
        s_new = s + jnp.sum(hist_v[pl.ds(b * _L, _L)])
        hit = (s_new >= _K) & (s < _K)
        return (
            s_new,
            jnp.where(hit, b, b_win),
            jnp.where(hit, s, above),
        )

    _, b_win, above = plsc.parallel_loop(
        0, _NB, unroll=4,
        carry=(jnp.int32(0), jnp.int32(0), jnp.int32(0)),
    )(scan_body)
    kprime = _K - above
    bwin_v = jnp.full((_L,), b_win, dtype=jnp.int32)

    def compact_body(i, off_c):
        v = data_v[pl.ds(i * _L, _L)]
        m = (v * 256.0).astype(jnp.int32) == bwin_v
        m = m & jnp.full((_L,), off_c < _CCAP - _L, dtype=jnp.bool_)
        plsc.store_compressed(cand_v.at[pl.ds(off_c, _L)], v, mask=m)
        return off_c + plsc.all_reduce_population_count(m)[0]

    off = plsc.parallel_loop(0, nvec, unroll=8, carry=jnp.int32(0))(
        compact_body
    )
    nvc = (off + _L - 1) // _L

    def round_body(r, lohi):
        lo, hi = lohi
        mid = (lo + hi) // 2
        tvec = jnp.full(
            (_L,), lax.bitcast_convert_type(mid, jnp.float32), dtype=jnp.float32
        )

        def cnt(i, a):
            v = cand_v[pl.ds(i * _L, _L)]
            return a + plsc.all_reduce_population_count(v >= tvec)

        acc = lax.fori_loop(0, nvc, cnt, zeros_i)
        plsc.fetch_and_add(cnt_sm.at[r], acc[0], subcore_id=0)
        plsc.subcore_barrier()
        tot = plsc.fetch_and_add(cnt_sm.at[r], jnp.int32(0), subcore_id=0)
        ok = tot >= kprime
        return jnp.where(ok, mid, lo), jnp.where(ok, hi, mid)

    lo0 = jnp.where(
        b_win > 0,
        lax.bitcast_convert_type(b_win.astype(jnp.float32) * (1.0 / 256.0),
                                 jnp.int32),
        jnp.int32(0),
    )
    hi0 = lax.bitcast_convert_type(
        (b_win + 1).astype(jnp.float32) * (1.0 / 256.0), jnp.int32
    )
    nrounds = (
        lax.shift_right_logical(
            lax.bitcast_convert_type((hi0 - lo0).astype(jnp.float32), jnp.int32),
            23,
        )
        - 126
    )
    lo, _ = lax.fori_loop(0, nrounds, round_body, (lo0, hi0))
    tau = jnp.full(
        (_L,),
        lax.bitcast_convert_type(lo, jnp.float32) * 0.25,
        dtype=jnp.float32,
    )

    @plsc.parallel_loop(0, nvec, unroll=8)
    def _mask(i):
        v = data_v[pl.ds(i * _L, _L)]
        data_v[pl.ds(i * _L, _L)] = jnp.where(v * 0.25 >= tau, 1.0, 0.0)

    @pl.when(tid == _NTILES - 1)
    def _store_tail():
        pltpu.sync_copy(data_v.at[pl.ds(0, _TAIL)], out_hbm.at[pl.ds(base, _TAIL)])

    @pl.when(tid != _NTILES - 1)
    def _store_full():
        pltpu.sync_copy(data_v, out_hbm.at[pl.ds(base, _CHUNK)])


def kernel(strength):
    x = strength.reshape(-1)
    m = _topk_mask(x)
    return (m != 0.0).reshape(_N, 1)
